# baseline (speedup 1.0000x reference)
import functools

import jax
import jax.numpy as jnp
from jax import lax
from jax.experimental import pallas as pl
from jax.experimental.pallas import tpu as pltpu
from jax.experimental.pallas import tpu_sc as plsc

_BETA_SQUARED = 1.0

_NC = 2
_NS = 16
_NW = _NC * _NS
_L = 16

_E = 16 * 1 * 512 * 512
_W = _E // _NW
_CHUNK = 16384
_NCHUNK = _W // _CHUNK


def _sc_partials_kernel(out_hbm, tgt_hbm, part_hbm,
                        ob0, ob1, tb0, tb1, pbuf,
                        so0, so1, st0, st1):
    wid = lax.axis_index("s") * _NC + lax.axis_index("c")
    base = wid * _W

    obufs = (ob0, ob1)
    tbufs = (tb0, tb1)
    osems = (so0, so1)
    tsems = (st0, st1)

    def start(g, slot):
        off = base + g * _CHUNK
        ho = pltpu.async_copy(out_hbm.at[pl.ds(off, _CHUNK)], obufs[slot],
                              osems[slot])
        ht = pltpu.async_copy(tgt_hbm.at[pl.ds(off, _CHUNK)], tbufs[slot],
                              tsems[slot])
        return ho, ht

    def make_chunk_body(slot):
        def chunk_body(j, accs):
            acc_tp, acc_so, acc_st = accs
            o = obufs[slot][pl.ds(j * _L, _L)]
            t = tbufs[slot][pl.ds(j * _L, _L)]
            m = o >= 0.5
            acc_so = acc_so + jnp.where(m, 1.0, 0.0)
            acc_st = acc_st + t
            acc_tp = acc_tp + jnp.where(m, t, 0.0)
            return acc_tp, acc_so, acc_st
        return chunk_body

    zeros = jnp.zeros((_L,), jnp.float32)
    acc_tp, acc_so, acc_st = zeros, zeros, zeros

    handles = [None, None]
    handles[0] = start(0, 0)
    for g in range(_NCHUNK):
        slot = g % 2
        if g + 1 < _NCHUNK:
            handles[1 - slot] = start(g + 1, 1 - slot)
        ho, ht = handles[slot]
        ho.wait()
        ht.wait()
        acc_tp, acc_so, acc_st = lax.fori_loop(
            0, _CHUNK // _L, make_chunk_body(slot), (acc_tp, acc_so, acc_st),
            unroll=8)

    pbuf[pl.ds(0, _L)] = acc_tp
    pbuf[pl.ds(_L, _L)] = acc_so
    pbuf[pl.ds(2 * _L, _L)] = acc_st
    pltpu.sync_copy(pbuf, part_hbm.at[wid])


def _finish_kernel(p_ref, o_ref):
    x = p_ref[...]
    tp = jnp.sum(x[:, 0:_L])
    sum_out = jnp.sum(x[:, _L:2 * _L])
    sum_tgt = jnp.sum(x[:, 2 * _L:3 * _L])
    fn = sum_tgt - tp
    fp = sum_out - tp
    recall = tp / (tp + fn)
    precision = tp / (tp + fp)
    f = ((1.0 + _BETA_SQUARED) * (precision * recall)
         / (_BETA_SQUARED * precision + recall))
    o_ref[...] = jnp.full((1, 1), f, jnp.float32)


def kernel(outputs, targets):
    out_flat = outputs.reshape(-1)
    tgt_flat = targets.reshape(-1)

    mesh = plsc.VectorSubcoreMesh(core_axis_name="c", subcore_axis_name="s",
                                  num_cores=_NC, num_subcores=_NS)
    partials = pl.kernel(
        _sc_partials_kernel,
        out_type=jax.ShapeDtypeStruct((_NW, 3 * _L), jnp.float32),
        mesh=mesh,
        scratch_types=[
            pltpu.VMEM((_CHUNK,), jnp.float32),
            pltpu.VMEM((_CHUNK,), jnp.float32),
            pltpu.VMEM((_CHUNK,), jnp.float32),
            pltpu.VMEM((_CHUNK,), jnp.float32),
            pltpu.VMEM((3 * _L,), jnp.float32),
            pltpu.SemaphoreType.DMA,
            pltpu.SemaphoreType.DMA,
            pltpu.SemaphoreType.DMA,
            pltpu.SemaphoreType.DMA,
        ],
    )(out_flat, tgt_flat)

    f = pl.pallas_call(
        _finish_kernel,
        out_shape=jax.ShapeDtypeStruct((1, 1), jnp.float32),
    )(partials)
    return f.reshape(())


# speedup vs baseline: 1.8591x; 1.8591x over previous
import functools

import jax
import jax.numpy as jnp
from jax import lax
from jax.experimental import pallas as pl
from jax.experimental.pallas import tpu as pltpu
from jax.experimental.pallas import tpu_sc as plsc

_BETA_SQUARED = 1.0

_NC = 2
_NS = 16
_NW = _NC * _NS
_L = 16

_B = 16
_R = 512
_C = 512
_E = _B * _R * _C
_W = _E // _NW
_WROWS = _R // (_NW // _B)
_CROWS = 32
_CHUNK = _CROWS * _C
_NCHUNK = _WROWS // _CROWS


def _sc_partials_kernel(out_hbm, tgt_hbm, part_hbm,
                        ob0, ob1, tb0, tb1, pbuf,
                        so0, so1, st0, st1):
    wid = lax.axis_index("s") * _NC + lax.axis_index("c")
    b = wid // 2
    row_base = (wid % 2) * _WROWS

    obufs = (ob0, ob1)
    tbufs = (tb0, tb1)
    osems = (so0, so1)
    tsems = (st0, st1)

    def start(g, slot):
        r0 = row_base + g * _CROWS
        ho = pltpu.async_copy(out_hbm.at[b, 0, pl.ds(r0, _CROWS), :],
                              obufs[slot], osems[slot])
        ht = pltpu.async_copy(tgt_hbm.at[b, 0, pl.ds(r0, _CROWS), :],
                              tbufs[slot], tsems[slot])
        return ho, ht

    def make_chunk_body(slot):
        def chunk_body(j, accs):
            acc_tp, acc_so, acc_st = accs
            for k in range(_C // _L):
                o = obufs[slot][j, pl.ds(k * _L, _L)]
                t = tbufs[slot][j, pl.ds(k * _L, _L)]
                m = o >= 0.5
                acc_so = acc_so + jnp.where(m, 1.0, 0.0)
                acc_st = acc_st + t
                acc_tp = acc_tp + jnp.where(m, t, 0.0)
            return acc_tp, acc_so, acc_st
        return chunk_body

    zeros = jnp.zeros((_L,), jnp.float32)
    acc_tp, acc_so, acc_st = zeros, zeros, zeros

    handles = [None, None]
    handles[0] = start(0, 0)
    for g in range(_NCHUNK):
        slot = g % 2
        if g + 1 < _NCHUNK:
            handles[1 - slot] = start(g + 1, 1 - slot)
        ho, ht = handles[slot]
        ho.wait()
        ht.wait()
        acc_tp, acc_so, acc_st = lax.fori_loop(
            0, _CROWS, make_chunk_body(slot), (acc_tp, acc_so, acc_st))

    pbuf[pl.ds(0, _L)] = acc_tp
    pbuf[pl.ds(_L, _L)] = acc_so
    pbuf[pl.ds(2 * _L, _L)] = acc_st
    pltpu.sync_copy(pbuf, part_hbm.at[wid])


def _finish_kernel(p_ref, o_ref):
    x = p_ref[...]
    tp = jnp.sum(x[:, 0:_L])
    sum_out = jnp.sum(x[:, _L:2 * _L])
    sum_tgt = jnp.sum(x[:, 2 * _L:3 * _L])
    fn = sum_tgt - tp
    fp = sum_out - tp
    recall = tp / (tp + fn)
    precision = tp / (tp + fp)
    f = ((1.0 + _BETA_SQUARED) * (precision * recall)
         / (_BETA_SQUARED * precision + recall))
    o_ref[...] = jnp.full((1, 1), f, jnp.float32)


def kernel(outputs, targets):
    mesh = plsc.VectorSubcoreMesh(core_axis_name="c", subcore_axis_name="s",
                                  num_cores=_NC, num_subcores=_NS)
    partials = pl.kernel(
        _sc_partials_kernel,
        out_type=jax.ShapeDtypeStruct((_NW, 3 * _L), jnp.float32),
        mesh=mesh,
        scratch_types=[
            pltpu.VMEM((_CROWS, _C), jnp.float32),
            pltpu.VMEM((_CROWS, _C), jnp.float32),
            pltpu.VMEM((_CROWS, _C), jnp.float32),
            pltpu.VMEM((_CROWS, _C), jnp.float32),
            pltpu.VMEM((3 * _L,), jnp.float32),
            pltpu.SemaphoreType.DMA,
            pltpu.SemaphoreType.DMA,
            pltpu.SemaphoreType.DMA,
            pltpu.SemaphoreType.DMA,
        ],
    )(outputs, targets)

    f = pl.pallas_call(
        _finish_kernel,
        out_shape=jax.ShapeDtypeStruct((1, 1), jnp.float32),
    )(partials)
    return f.reshape(())


# speedup vs baseline: 1.8783x; 1.0104x over previous
import functools

import jax
import jax.numpy as jnp
from jax import lax
from jax.experimental import pallas as pl
from jax.experimental.pallas import tpu as pltpu
from jax.experimental.pallas import tpu_sc as plsc

_BETA_SQUARED = 1.0

_NC = 2
_NS = 16
_NW = _NC * _NS
_L = 16

_C = 512
_ROWS = 8192

_SC_ROWS = 2560
_PW = _SC_ROWS // _NW
_NPH = 2
_PH_ROWS = _PW // _NPH

_TC_ROWS = _ROWS - _SC_ROWS
_TC_BLK = 512
_TC_GRID = _TC_ROWS // _TC_BLK


def _sc_partials_kernel(o_hbm, t_hbm, part_hbm,
                        ob0, tb0, ob1, tb1, pbuf,
                        so0, st0, so1, st1):
    wid = lax.axis_index("s") * _NC + lax.axis_index("c")
    r0 = wid * _PW

    obufs = (ob0, ob1)
    tbufs = (tb0, tb1)
    osems = (so0, so1)
    tsems = (st0, st1)

    handles = []
    for ph in range(_NPH):
        r = r0 + ph * _PH_ROWS
        ho = pltpu.async_copy(o_hbm.at[pl.ds(r, _PH_ROWS), :], obufs[ph],
                              osems[ph])
        ht = pltpu.async_copy(t_hbm.at[pl.ds(r, _PH_ROWS), :], tbufs[ph],
                              tsems[ph])
        handles.append((ho, ht))

    def make_row_body(ph):
        def row_body(j, accs):
            acc_tp, acc_so, acc_st = accs
            for k in range(_C // _L):
                o = obufs[ph][j, pl.ds(k * _L, _L)]
                t = tbufs[ph][j, pl.ds(k * _L, _L)]
                m = o >= 0.5
                acc_so = acc_so + jnp.where(m, 1.0, 0.0)
                acc_st = acc_st + t
                acc_tp = acc_tp + jnp.where(m, t, 0.0)
            return acc_tp, acc_so, acc_st
        return row_body

    zeros = jnp.zeros((_L,), jnp.float32)
    accs = (zeros, zeros, zeros)
    for ph in range(_NPH):
        ho, ht = handles[ph]
        ho.wait()
        ht.wait()
        accs = lax.fori_loop(0, _PH_ROWS, make_row_body(ph), accs)

    acc_tp, acc_so, acc_st = accs
    pbuf[pl.ds(0, _L)] = acc_tp
    pbuf[pl.ds(_L, _L)] = acc_so
    pbuf[pl.ds(2 * _L, _L)] = acc_st
    pltpu.sync_copy(pbuf, part_hbm.at[wid])


def _tc_partials_kernel(o_ref, t_ref, acc_ref):
    i = pl.program_id(0)

    @pl.when(i == 0)
    def _():
        acc_ref[...] = jnp.zeros_like(acc_ref)

    o = o_ref[...]
    t = t_ref[...]
    m = o >= 0.5
    ob = jnp.where(m, 1.0, 0.0)
    tpv = jnp.where(m, t, 0.0)

    def red(v):
        return jnp.sum(v.reshape(_TC_BLK // 8, 8, _C // 128, 128),
                       axis=(0, 2))

    acc_ref[0] += red(tpv)
    acc_ref[1] += red(ob)
    acc_ref[2] += red(t)


def _finish_kernel(sc_ref, tc_ref, o_ref):
    s = sc_ref[...]
    c = tc_ref[...]
    tp = jnp.sum(s[:, 0:_L]) + jnp.sum(c[0])
    sum_out = jnp.sum(s[:, _L:2 * _L]) + jnp.sum(c[1])
    sum_tgt = jnp.sum(s[:, 2 * _L:3 * _L]) + jnp.sum(c[2])
    fn = sum_tgt - tp
    fp = sum_out - tp
    recall = tp / (tp + fn)
    precision = tp / (tp + fp)
    f = ((1.0 + _BETA_SQUARED) * (precision * recall)
         / (_BETA_SQUARED * precision + recall))
    o_ref[...] = jnp.full((1, 1), f, jnp.float32)


def kernel(outputs, targets):
    o2 = outputs.reshape(_ROWS, _C)
    t2 = targets.reshape(_ROWS, _C)

    mesh = plsc.VectorSubcoreMesh(core_axis_name="c", subcore_axis_name="s",
                                  num_cores=_NC, num_subcores=_NS)
    sc_partials = pl.kernel(
        _sc_partials_kernel,
        out_type=jax.ShapeDtypeStruct((_NW, 3 * _L), jnp.float32),
        mesh=mesh,
        scratch_types=[
            pltpu.VMEM((_PH_ROWS, _C), jnp.float32),
            pltpu.VMEM((_PH_ROWS, _C), jnp.float32),
            pltpu.VMEM((_PH_ROWS, _C), jnp.float32),
            pltpu.VMEM((_PH_ROWS, _C), jnp.float32),
            pltpu.VMEM((3 * _L,), jnp.float32),
            pltpu.SemaphoreType.DMA,
            pltpu.SemaphoreType.DMA,
            pltpu.SemaphoreType.DMA,
            pltpu.SemaphoreType.DMA,
        ],
    )(o2, t2)

    tc_partials = pl.pallas_call(
        _tc_partials_kernel,
        grid=(_TC_GRID,),
        in_specs=[
            pl.BlockSpec((_TC_BLK, _C),
                         lambda i: (i + _SC_ROWS // _TC_BLK, 0)),
            pl.BlockSpec((_TC_BLK, _C),
                         lambda i: (i + _SC_ROWS // _TC_BLK, 0)),
        ],
        out_specs=pl.BlockSpec((3, 8, 128), lambda i: (0, 0, 0)),
        out_shape=jax.ShapeDtypeStruct((3, 8, 128), jnp.float32),
    )(o2, t2)

    f = pl.pallas_call(
        _finish_kernel,
        out_shape=jax.ShapeDtypeStruct((1, 1), jnp.float32),
    )(sc_partials, tc_partials)
    return f.reshape(())
